# baseline (speedup 1.0000x reference)
import functools

import jax
import jax.numpy as jnp
from jax import lax
from jax.experimental import pallas as pl
from jax.experimental.pallas import tpu as pltpu
from jax.experimental.pallas import tpu_sc as plsc

F32 = jnp.float32
I32 = jnp.int32

NC = 2
NS = 16
NW = NC * NS
LANES = 16
CHUNK = 16
L_NODES = 32
LAT = 64
ROW = L_NODES * LAT


def _qproj(x, W, b, blk=1024):
    T, D = x.shape
    O = W.shape[1]

    def body(x_ref, w_ref, b_ref, o_ref):
        o_ref[...] = (
            jnp.dot(x_ref[...], w_ref[...], preferred_element_type=F32)
            + b_ref[...]
        )

    return pl.pallas_call(
        body,
        grid=(T // blk,),
        in_specs=[
            pl.BlockSpec((blk, D), lambda i: (i, 0)),
            pl.BlockSpec((D, O), lambda i: (0, 0)),
            pl.BlockSpec((1, O), lambda i: (0, 0)),
        ],
        out_specs=pl.BlockSpec((blk, O), lambda i: (i, 0)),
        out_shape=jax.ShapeDtypeStruct((T, O), F32),
    )(x, W, b.reshape(1, O))


SUPER = 8
STOK = SUPER * CHUNK


def _attention_sc(table, batch_idx, q):
    T = batch_idx.shape[0]
    tpw = T // NW
    n_super = tpw // STOK

    mesh = plsc.VectorSubcoreMesh(
        core_axis_name="c", subcore_axis_name="s",
        num_cores=NC, num_subcores=NS)

    @functools.partial(
        pl.kernel,
        out_type=jax.ShapeDtypeStruct((T, LAT), F32),
        mesh=mesh,
        compiler_params=pltpu.CompilerParams(needs_layout_passes=False),
        scratch_types=[
            pltpu.VMEM((STOK,), I32),
            pltpu.VMEM((STOK, LAT), F32),
            pltpu.VMEM((STOK, LAT), F32),
            pltpu.VMEM((CHUNK, ROW), F32),
            pltpu.VMEM((CHUNK, ROW), F32),
            pltpu.VMEM((L_NODES, LANES), F32),
            pltpu.SemaphoreType.DMA,
            pltpu.SemaphoreType.DMA,
        ],
    )
    def body(tab_hbm, idx_hbm, q_hbm, out_hbm, idx_big, q_big, cxt_big,
             slabA, slabB, pbuf, semA, semB):
        wid = lax.axis_index("s") * NC + lax.axis_index("c")
        lane = lax.iota(I32, LANES)

        def compute_chunk(c, slab, pbuf):
            def one_token(t):
                row = c * CHUNK + t
                qr = [q_big[row, pl.ds(dc * 16, 16)] for dc in range(4)]

                sv = [jnp.zeros((LANES,), F32), jnp.zeros((LANES,), F32)]
                for h in range(2):
                    for j in range(16):
                        l = h * 16 + j
                        acc = slab[t, pl.ds(l * LAT, 16)] * qr[0]
                        for dc in range(1, 4):
                            acc += slab[t, pl.ds(l * LAT + dc * 16, 16)] * qr[dc]
                        s = jnp.sum(acc)
                        sv[h] = jnp.where(lane == j, s, sv[h])

                m = jnp.full((LANES,), jnp.max(jnp.maximum(sv[0], sv[1])), F32)
                e = [jnp.exp(sv[0] - m), jnp.exp(sv[1] - m)]
                inv = 1.0 / jnp.full((LANES,), jnp.sum(e[0] + e[1]), F32)

                w = [jnp.zeros((LANES,), F32) for _ in range(4)]
                for h in range(2):
                    for j in range(16):
                        l = h * 16 + j
                        pj = e[h][j]
                        for dc in range(4):
                            w[dc] += pj * slab[t, pl.ds(l * LAT + dc * 16, 16)]
                for dc in range(4):
                    cxt_big[row, pl.ds(dc * 16, 16)] = w[dc] * inv

            def tok_pair(p, _):
                one_token(2 * p)
                one_token(2 * p + 1)
                return 0

            lax.fori_loop(0, CHUNK // 2, tok_pair, 0)

        def super_body(s, _):
            sbase = wid * tpw + s * STOK
            pltpu.sync_copy(idx_hbm.at[pl.ds(sbase, STOK)], idx_big)
            pltpu.sync_copy(q_hbm.at[pl.ds(sbase, STOK)], q_big)
            pltpu.async_copy(
                tab_hbm.at[idx_big.at[pl.ds(0, CHUNK)]], slabA, semA)

            def pair_body(p, _):
                cA = 2 * p
                cB = 2 * p + 1
                pltpu.async_copy(
                    tab_hbm.at[idx_big.at[pl.ds(cB * CHUNK, CHUNK)]],
                    slabB, semB)
                pltpu.make_async_copy(
                    tab_hbm.at[idx_big.at[pl.ds(cA * CHUNK, CHUNK)]],
                    slabA, semA).wait()
                compute_chunk(cA, slabA, pbuf)

                @pl.when(cB + 1 < SUPER)
                def _():
                    pltpu.async_copy(
                        tab_hbm.at[idx_big.at[pl.ds((cB + 1) * CHUNK, CHUNK)]],
                        slabA, semA)
                pltpu.make_async_copy(
                    tab_hbm.at[idx_big.at[pl.ds(cB * CHUNK, CHUNK)]],
                    slabB, semB).wait()
                compute_chunk(cB, slabB, pbuf)
                return 0

            lax.fori_loop(0, SUPER // 2, pair_body, 0)
            pltpu.sync_copy(cxt_big, out_hbm.at[pl.ds(sbase, STOK)])
            return 0

        lax.fori_loop(0, n_super, super_body, 0)

    return body(table, batch_idx, q)


def _loss_ce(x, cxt, labels, W1a, W1b, b1, W2, b2, coff=0, blk=1024):
    T, D = x.shape
    V = W2.shape[1]
    nb = T // blk
    labels3 = labels.reshape(nb, 1, blk)

    def body(x_ref, c_ref, w1a_ref, w1b_ref, b1_ref, w2_ref, b2_ref,
             lab_ref, o_ref):
        i = pl.program_id(0)
        h = jnp.dot(x_ref[...], w1a_ref[...], preferred_element_type=F32)
        h += jnp.dot(c_ref[...], w1b_ref[...], preferred_element_type=F32)
        h = jnp.maximum(h + b1_ref[...], 0.0)
        logits = jnp.dot(h, w2_ref[...], preferred_element_type=F32) + b2_ref[...]
        m = jnp.max(logits, axis=-1, keepdims=True)
        lse = jnp.log(jnp.sum(jnp.exp(logits - m), axis=-1)) + m[:, 0]
        lab = lab_ref[0, 0, :]
        onehot = lax.broadcasted_iota(I32, (blk, V), 1) == lab[:, None]
        picked = jnp.sum(jnp.where(onehot, logits, 0.0), axis=-1)
        part = jnp.sum(lse - picked)

        @pl.when(i == 0)
        def _():
            o_ref[0, 0] = 0.0

        o_ref[0, 0] += part

    out = pl.pallas_call(
        body,
        grid=(nb,),
        in_specs=[
            pl.BlockSpec((blk, D), lambda i: (i, 0)),
            pl.BlockSpec((blk, LAT), lambda i: (coff + i, 0)),
            pl.BlockSpec(W1a.shape, lambda i: (0, 0)),
            pl.BlockSpec(W1b.shape, lambda i: (0, 0)),
            pl.BlockSpec((1, W1a.shape[1]), lambda i: (0, 0)),
            pl.BlockSpec(W2.shape, lambda i: (0, 0)),
            pl.BlockSpec((1, V), lambda i: (0, 0)),
            pl.BlockSpec((1, 1, blk), lambda i: (i, 0, 0)),
        ],
        out_specs=pl.BlockSpec(memory_space=pltpu.SMEM),
        out_shape=jax.ShapeDtypeStruct((1, 1), F32),
    )(x, cxt, W1a, W1b, b1.reshape(1, -1), W2, b2.reshape(1, V), labels3)
    return out[0, 0]


def _loss_bce(x, cxt, labels, W1a, W1b, b1, W2, b2, coff=0, blk=1024):
    T, D = x.shape
    nb = T // blk
    labels3 = labels.reshape(nb, 1, blk)

    def body(x_ref, c_ref, w1a_ref, w1b_ref, b1_ref, w2_ref, b2_ref,
             lab_ref, o_ref):
        i = pl.program_id(0)
        h = jnp.dot(x_ref[...], w1a_ref[...], preferred_element_type=F32)
        h += jnp.dot(c_ref[...], w1b_ref[...], preferred_element_type=F32)
        h = jnp.maximum(h + b1_ref[...], 0.0)
        s = jnp.dot(h, w2_ref[...], preferred_element_type=F32)[:, 0] + b2_ref[0, 0]
        y = lab_ref[0, 0, :].astype(F32)
        part = jnp.sum(
            jnp.maximum(s, 0.0) - s * y + jnp.log(1.0 + jnp.exp(-jnp.abs(s)))
        )

        @pl.when(i == 0)
        def _():
            o_ref[0, 0] = 0.0

        o_ref[0, 0] += part

    out = pl.pallas_call(
        body,
        grid=(nb,),
        in_specs=[
            pl.BlockSpec((blk, D), lambda i: (i, 0)),
            pl.BlockSpec((blk, LAT), lambda i: (coff + i, 0)),
            pl.BlockSpec(W1a.shape, lambda i: (0, 0)),
            pl.BlockSpec(W1b.shape, lambda i: (0, 0)),
            pl.BlockSpec((1, W1a.shape[1]), lambda i: (0, 0)),
            pl.BlockSpec(W2.shape, lambda i: (0, 0)),
            pl.BlockSpec((1, 1), lambda i: (0, 0)),
            pl.BlockSpec((1, 1, blk), lambda i: (i, 0, 0)),
        ],
        out_specs=pl.BlockSpec(memory_space=pltpu.SMEM),
        out_shape=jax.ShapeDtypeStruct((1, 1), F32),
    )(x, cxt, W1a, W1b, b1.reshape(1, -1), W2, b2.reshape(1, 1), labels3)
    return out[0, 0]


def kernel(src_graph_vecs, topo_vecs, atom_vecs, bond_vecs,
           A_topo_W, A_topo_b, A_atom_W, A_atom_b, A_bond_W, A_bond_b,
           topo_W1, topo_b1, topo_W2, topo_b2,
           atom_W1, atom_b1, atom_W2, atom_b2,
           bond_W1, bond_b1, bond_W2, bond_b2,
           batch_idx_topo, batch_idx_atom, batch_idx_bond,
           topo_labels, atom_labels, bond_labels):
    B = src_graph_vecs.shape[0]
    table = src_graph_vecs.reshape(B, ROW)

    q_topo = _qproj(topo_vecs, A_topo_W, A_topo_b)
    q_atom = _qproj(atom_vecs, A_atom_W, A_atom_b)
    q_bond = _qproj(bond_vecs, A_bond_W, A_bond_b)

    Tt = batch_idx_topo.shape[0]
    Ta = batch_idx_atom.shape[0]
    idx_all = jnp.concatenate([batch_idx_topo, batch_idx_atom, batch_idx_bond])
    q_all = jnp.concatenate([q_topo, q_atom, q_bond], axis=0)
    cxt_all = _attention_sc(table, idx_all, q_all)

    Dt = topo_vecs.shape[1]
    Da = atom_vecs.shape[1]
    Db = bond_vecs.shape[1]
    lt = _loss_bce(topo_vecs, cxt_all, topo_labels,
                   topo_W1[:Dt], topo_W1[Dt:], topo_b1, topo_W2, topo_b2,
                   coff=0)
    la = _loss_ce(atom_vecs, cxt_all, atom_labels,
                  atom_W1[:Da], atom_W1[Da:], atom_b1, atom_W2, atom_b2,
                  coff=Tt // 1024)
    lb = _loss_ce(bond_vecs, cxt_all, bond_labels,
                  bond_W1[:Db], bond_W1[Db:], bond_b1, bond_W2, bond_b2,
                  coff=(Tt + Ta) // 1024)

    return (lt + la + lb) / B


# speedup vs baseline: 1.3901x; 1.3901x over previous
import functools

import jax
import jax.numpy as jnp
from jax import lax
from jax.experimental import pallas as pl
from jax.experimental.pallas import tpu as pltpu
from jax.experimental.pallas import tpu_sc as plsc

F32 = jnp.float32
I32 = jnp.int32

NC = 2
NS = 16
NW = NC * NS
LANES = 16
CHUNK = 16
L_NODES = 32
LAT = 64
ROW = L_NODES * LAT


def _qproj(x, W, b, blk=1024):
    T, D = x.shape
    O = W.shape[1]

    def body(x_ref, w_ref, b_ref, o_ref):
        o_ref[...] = (
            jnp.dot(x_ref[...], w_ref[...], preferred_element_type=F32)
            + b_ref[...]
        )

    return pl.pallas_call(
        body,
        grid=(T // blk,),
        in_specs=[
            pl.BlockSpec((blk, D), lambda i: (i, 0)),
            pl.BlockSpec((D, O), lambda i: (0, 0)),
            pl.BlockSpec((1, O), lambda i: (0, 0)),
        ],
        out_specs=pl.BlockSpec((blk, O), lambda i: (i, 0)),
        out_shape=jax.ShapeDtypeStruct((T, O), F32),
    )(x, W, b.reshape(1, O))


SUPER = 8
STOK = SUPER * CHUNK


def _attention_sc(table, batch_idx, q):
    T = batch_idx.shape[0]
    tpw = T // NW
    n_super = tpw // STOK

    mesh = plsc.VectorSubcoreMesh(
        core_axis_name="c", subcore_axis_name="s",
        num_cores=NC, num_subcores=NS)

    @functools.partial(
        pl.kernel,
        out_type=jax.ShapeDtypeStruct((T, LAT), F32),
        mesh=mesh,
        compiler_params=pltpu.CompilerParams(needs_layout_passes=False),
        scratch_types=[
            pltpu.VMEM((STOK,), I32),
            pltpu.VMEM((STOK,), I32),
            pltpu.VMEM((STOK, LAT), F32),
            pltpu.VMEM((STOK, LAT), F32),
            pltpu.VMEM((STOK, LAT), F32),
            pltpu.VMEM((CHUNK, ROW), F32),
            pltpu.VMEM((CHUNK, ROW), F32),
            pltpu.SemaphoreType.DMA,
            pltpu.SemaphoreType.DMA,
            pltpu.SemaphoreType.DMA,
            pltpu.SemaphoreType.DMA,
        ],
    )
    def body(tab_hbm, idx_hbm, q_hbm, out_hbm, idx0, idx1, q0, q1, cxt_big,
             slabA, slabB, semA, semB, semP0, semP1):
        wid = lax.axis_index("s") * NC + lax.axis_index("c")
        lane = lax.iota(I32, LANES)

        def compute_chunk(c, slab, q_big):
            def one_token(t):
                row = c * CHUNK + t
                qr = [q_big[row, pl.ds(dc * 16, 16)] for dc in range(4)]

                sv = [jnp.zeros((LANES,), F32), jnp.zeros((LANES,), F32)]
                for h in range(2):
                    for j in range(16):
                        l = h * 16 + j
                        acc = slab[t, pl.ds(l * LAT, 16)] * qr[0]
                        for dc in range(1, 4):
                            acc += slab[t, pl.ds(l * LAT + dc * 16, 16)] * qr[dc]
                        s = jnp.sum(acc)
                        sv[h] = jnp.where(lane == j, s, sv[h])

                m = jnp.full((LANES,), jnp.max(jnp.maximum(sv[0], sv[1])), F32)
                e = [jnp.exp(sv[0] - m), jnp.exp(sv[1] - m)]
                inv = 1.0 / jnp.full((LANES,), jnp.sum(e[0] + e[1]), F32)

                w = [jnp.zeros((LANES,), F32) for _ in range(4)]
                for h in range(2):
                    for j in range(16):
                        l = h * 16 + j
                        pj = e[h][j]
                        for dc in range(4):
                            w[dc] += pj * slab[t, pl.ds(l * LAT + dc * 16, 16)]
                for dc in range(4):
                    cxt_big[row, pl.ds(dc * 16, 16)] = w[dc] * inv

            def tok_pair(p, _):
                one_token(2 * p)
                one_token(2 * p + 1)
                return 0

            lax.fori_loop(0, CHUNK // 2, tok_pair, 0)

        def run_super(s, idxb, qb, nidxb, nqb, nsemP):
            sbase = wid * tpw + s * STOK
            ns = s + 1
            nsbase = wid * tpw + jnp.where(ns < n_super, ns, 0) * STOK
            pltpu.async_copy(idx_hbm.at[pl.ds(nsbase, STOK)], nidxb, nsemP)
            pltpu.async_copy(q_hbm.at[pl.ds(nsbase, STOK)], nqb, nsemP)

            def pair_body(p, _):
                cA = 2 * p
                cB = 2 * p + 1
                pltpu.async_copy(
                    tab_hbm.at[idxb.at[pl.ds(cB * CHUNK, CHUNK)]],
                    slabB, semB)
                pltpu.make_async_copy(
                    tab_hbm.at[idxb.at[pl.ds(cA * CHUNK, CHUNK)]],
                    slabA, semA).wait()
                compute_chunk(cA, slabA, qb)

                @pl.when(cB + 1 < SUPER)
                def _():
                    pltpu.async_copy(
                        tab_hbm.at[idxb.at[pl.ds((cB + 1) * CHUNK, CHUNK)]],
                        slabA, semA)

                @pl.when(cB + 1 >= SUPER)
                def _():
                    pltpu.make_async_copy(
                        idx_hbm.at[pl.ds(nsbase, STOK)], nidxb, nsemP).wait()
                    pltpu.make_async_copy(
                        q_hbm.at[pl.ds(nsbase, STOK)], nqb, nsemP).wait()
                    pltpu.async_copy(
                        tab_hbm.at[nidxb.at[pl.ds(0, CHUNK)]], slabA, semA)

                pltpu.make_async_copy(
                    tab_hbm.at[idxb.at[pl.ds(cB * CHUNK, CHUNK)]],
                    slabB, semB).wait()
                compute_chunk(cB, slabB, qb)
                return 0

            lax.fori_loop(0, SUPER // 2, pair_body, 0)
            pltpu.sync_copy(cxt_big, out_hbm.at[pl.ds(sbase, STOK)])

        base0 = wid * tpw
        pltpu.sync_copy(idx_hbm.at[pl.ds(base0, STOK)], idx0)
        pltpu.sync_copy(q_hbm.at[pl.ds(base0, STOK)], q0)
        pltpu.async_copy(tab_hbm.at[idx0.at[pl.ds(0, CHUNK)]], slabA, semA)

        def spair_body(sp, _):
            run_super(2 * sp, idx0, q0, idx1, q1, semP1)
            run_super(2 * sp + 1, idx1, q1, idx0, q0, semP0)
            return 0

        lax.fori_loop(0, n_super // 2, spair_body, 0)
        pltpu.make_async_copy(
            tab_hbm.at[idx0.at[pl.ds(0, CHUNK)]], slabA, semA).wait()

    return body(table, batch_idx, q)


def _loss_ce(x, cxt, labels, W1a, W1b, b1, W2, b2, blk=1024):
    T, D = x.shape
    V = W2.shape[1]
    nb = T // blk
    labels3 = labels.reshape(nb, 1, blk)

    def body(x_ref, c_ref, w1a_ref, w1b_ref, b1_ref, w2_ref, b2_ref,
             lab_ref, o_ref):
        i = pl.program_id(0)
        h = jnp.dot(x_ref[...], w1a_ref[...], preferred_element_type=F32)
        h += jnp.dot(c_ref[...], w1b_ref[...], preferred_element_type=F32)
        h = jnp.maximum(h + b1_ref[...], 0.0)
        logits = jnp.dot(h, w2_ref[...], preferred_element_type=F32) + b2_ref[...]
        m = jnp.max(logits, axis=-1, keepdims=True)
        lse = jnp.log(jnp.sum(jnp.exp(logits - m), axis=-1)) + m[:, 0]
        lab = lab_ref[0, 0, :]
        onehot = lax.broadcasted_iota(I32, (blk, V), 1) == lab[:, None]
        picked = jnp.sum(jnp.where(onehot, logits, 0.0), axis=-1)
        part = jnp.sum(lse - picked)

        @pl.when(i == 0)
        def _():
            o_ref[0, 0] = 0.0

        o_ref[0, 0] += part

    out = pl.pallas_call(
        body,
        grid=(nb,),
        in_specs=[
            pl.BlockSpec((blk, D), lambda i: (i, 0)),
            pl.BlockSpec((blk, LAT), lambda i: (i, 0)),
            pl.BlockSpec(W1a.shape, lambda i: (0, 0)),
            pl.BlockSpec(W1b.shape, lambda i: (0, 0)),
            pl.BlockSpec((1, W1a.shape[1]), lambda i: (0, 0)),
            pl.BlockSpec(W2.shape, lambda i: (0, 0)),
            pl.BlockSpec((1, V), lambda i: (0, 0)),
            pl.BlockSpec((1, 1, blk), lambda i: (i, 0, 0)),
        ],
        out_specs=pl.BlockSpec(memory_space=pltpu.SMEM),
        out_shape=jax.ShapeDtypeStruct((1, 1), F32),
    )(x, cxt, W1a, W1b, b1.reshape(1, -1), W2, b2.reshape(1, V), labels3)
    return out[0, 0]


def _loss_bce(x, cxt, labels, W1a, W1b, b1, W2, b2, blk=1024):
    T, D = x.shape
    nb = T // blk
    labels3 = labels.reshape(nb, 1, blk)

    def body(x_ref, c_ref, w1a_ref, w1b_ref, b1_ref, w2_ref, b2_ref,
             lab_ref, o_ref):
        i = pl.program_id(0)
        h = jnp.dot(x_ref[...], w1a_ref[...], preferred_element_type=F32)
        h += jnp.dot(c_ref[...], w1b_ref[...], preferred_element_type=F32)
        h = jnp.maximum(h + b1_ref[...], 0.0)
        s = jnp.dot(h, w2_ref[...], preferred_element_type=F32)[:, 0] + b2_ref[0, 0]
        y = lab_ref[0, 0, :].astype(F32)
        part = jnp.sum(
            jnp.maximum(s, 0.0) - s * y + jnp.log(1.0 + jnp.exp(-jnp.abs(s)))
        )

        @pl.when(i == 0)
        def _():
            o_ref[0, 0] = 0.0

        o_ref[0, 0] += part

    out = pl.pallas_call(
        body,
        grid=(nb,),
        in_specs=[
            pl.BlockSpec((blk, D), lambda i: (i, 0)),
            pl.BlockSpec((blk, LAT), lambda i: (i, 0)),
            pl.BlockSpec(W1a.shape, lambda i: (0, 0)),
            pl.BlockSpec(W1b.shape, lambda i: (0, 0)),
            pl.BlockSpec((1, W1a.shape[1]), lambda i: (0, 0)),
            pl.BlockSpec(W2.shape, lambda i: (0, 0)),
            pl.BlockSpec((1, 1), lambda i: (0, 0)),
            pl.BlockSpec((1, 1, blk), lambda i: (i, 0, 0)),
        ],
        out_specs=pl.BlockSpec(memory_space=pltpu.SMEM),
        out_shape=jax.ShapeDtypeStruct((1, 1), F32),
    )(x, cxt, W1a, W1b, b1.reshape(1, -1), W2, b2.reshape(1, 1), labels3)
    return out[0, 0]


def kernel(src_graph_vecs, topo_vecs, atom_vecs, bond_vecs,
           A_topo_W, A_topo_b, A_atom_W, A_atom_b, A_bond_W, A_bond_b,
           topo_W1, topo_b1, topo_W2, topo_b2,
           atom_W1, atom_b1, atom_W2, atom_b2,
           bond_W1, bond_b1, bond_W2, bond_b2,
           batch_idx_topo, batch_idx_atom, batch_idx_bond,
           topo_labels, atom_labels, bond_labels):
    B = src_graph_vecs.shape[0]
    table = src_graph_vecs.reshape(B, ROW)

    q_topo = _qproj(topo_vecs, A_topo_W, A_topo_b)
    q_atom = _qproj(atom_vecs, A_atom_W, A_atom_b)
    q_bond = _qproj(bond_vecs, A_bond_W, A_bond_b)

    c_topo = _attention_sc(table, batch_idx_topo, q_topo)
    c_atom = _attention_sc(table, batch_idx_atom, q_atom)
    c_bond = _attention_sc(table, batch_idx_bond, q_bond)

    Dt = topo_vecs.shape[1]
    Da = atom_vecs.shape[1]
    Db = bond_vecs.shape[1]
    lt = _loss_bce(topo_vecs, c_topo, topo_labels,
                   topo_W1[:Dt], topo_W1[Dt:], topo_b1, topo_W2, topo_b2)
    la = _loss_ce(atom_vecs, c_atom, atom_labels,
                  atom_W1[:Da], atom_W1[Da:], atom_b1, atom_W2, atom_b2)
    lb = _loss_ce(bond_vecs, c_bond, bond_labels,
                  bond_W1[:Db], bond_W1[Db:], bond_b1, bond_W2, bond_b2)

    return (lt + la + lb) / B


# speedup vs baseline: 1.4280x; 1.0273x over previous
import functools

import jax
import jax.numpy as jnp
from jax import lax
from jax.experimental import pallas as pl
from jax.experimental.pallas import tpu as pltpu
from jax.experimental.pallas import tpu_sc as plsc

F32 = jnp.float32
I32 = jnp.int32

NC = 2
NS = 16
NW = NC * NS
LANES = 16
CHUNK = 16
L_NODES = 32
LAT = 64
ROW = L_NODES * LAT


def _qproj(x, W, b, blk=2048):
    T, D = x.shape
    O = W.shape[1]

    def body(x_ref, w_ref, b_ref, o_ref):
        o_ref[...] = (
            jnp.dot(x_ref[...], w_ref[...], preferred_element_type=F32)
            + b_ref[...]
        )

    return pl.pallas_call(
        body,
        grid=(T // blk,),
        in_specs=[
            pl.BlockSpec((blk, D), lambda i: (i, 0)),
            pl.BlockSpec((D, O), lambda i: (0, 0)),
            pl.BlockSpec((1, O), lambda i: (0, 0)),
        ],
        out_specs=pl.BlockSpec((blk, O), lambda i: (i, 0)),
        out_shape=jax.ShapeDtypeStruct((T, O), F32),
    )(x, W, b.reshape(1, O))


SUPER = 8
STOK = SUPER * CHUNK


def _attention_sc(table, batch_idx, q):
    T = batch_idx.shape[0]
    tpw = T // NW
    n_super = tpw // STOK

    mesh = plsc.VectorSubcoreMesh(
        core_axis_name="c", subcore_axis_name="s",
        num_cores=NC, num_subcores=NS)

    @functools.partial(
        pl.kernel,
        out_type=jax.ShapeDtypeStruct((T, LAT), F32),
        mesh=mesh,
        compiler_params=pltpu.CompilerParams(needs_layout_passes=False),
        scratch_types=[
            pltpu.VMEM((STOK,), I32),
            pltpu.VMEM((STOK,), I32),
            pltpu.VMEM((STOK, LAT), F32),
            pltpu.VMEM((STOK, LAT), F32),
            pltpu.VMEM((STOK, LAT), F32),
            pltpu.VMEM((CHUNK, ROW), F32),
            pltpu.VMEM((CHUNK, ROW), F32),
            pltpu.SemaphoreType.DMA,
            pltpu.SemaphoreType.DMA,
            pltpu.SemaphoreType.DMA,
            pltpu.SemaphoreType.DMA,
        ],
    )
    def body(tab_hbm, idx_hbm, q_hbm, out_hbm, idx0, idx1, q0, q1, cxt_big,
             slabA, slabB, semA, semB, semP0, semP1):
        wid = lax.axis_index("s") * NC + lax.axis_index("c")
        lane = lax.iota(I32, LANES)

        def compute_chunk(c, slab, q_big):
            def one_token(t):
                row = c * CHUNK + t
                qr = [q_big[row, pl.ds(dc * 16, 16)] for dc in range(4)]

                sv = [jnp.zeros((LANES,), F32), jnp.zeros((LANES,), F32)]
                for h in range(2):
                    for j in range(16):
                        l = h * 16 + j
                        acc = slab[t, pl.ds(l * LAT, 16)] * qr[0]
                        for dc in range(1, 4):
                            acc += slab[t, pl.ds(l * LAT + dc * 16, 16)] * qr[dc]
                        s = jnp.sum(acc)
                        sv[h] = jnp.where(lane == j, s, sv[h])

                m = jnp.full((LANES,), jnp.max(jnp.maximum(sv[0], sv[1])), F32)
                e = [jnp.exp(sv[0] - m), jnp.exp(sv[1] - m)]
                inv = 1.0 / jnp.full((LANES,), jnp.sum(e[0] + e[1]), F32)

                w = [jnp.zeros((LANES,), F32) for _ in range(4)]
                for h in range(2):
                    for j in range(16):
                        l = h * 16 + j
                        pj = e[h][j]
                        for dc in range(4):
                            w[dc] += pj * slab[t, pl.ds(l * LAT + dc * 16, 16)]
                for dc in range(4):
                    cxt_big[row, pl.ds(dc * 16, 16)] = w[dc] * inv

            def tok_pair(p, _):
                one_token(2 * p)
                one_token(2 * p + 1)
                return 0

            lax.fori_loop(0, CHUNK // 2, tok_pair, 0)

        def run_super(s, idxb, qb, nidxb, nqb, nsemP):
            sbase = wid * tpw + s * STOK
            ns = s + 1
            nsbase = wid * tpw + jnp.where(ns < n_super, ns, 0) * STOK
            pltpu.async_copy(idx_hbm.at[pl.ds(nsbase, STOK)], nidxb, nsemP)
            pltpu.async_copy(q_hbm.at[pl.ds(nsbase, STOK)], nqb, nsemP)

            def pair_body(p, _):
                cA = 2 * p
                cB = 2 * p + 1
                pltpu.async_copy(
                    tab_hbm.at[idxb.at[pl.ds(cB * CHUNK, CHUNK)]],
                    slabB, semB)
                pltpu.make_async_copy(
                    tab_hbm.at[idxb.at[pl.ds(cA * CHUNK, CHUNK)]],
                    slabA, semA).wait()
                compute_chunk(cA, slabA, qb)

                @pl.when(cB + 1 < SUPER)
                def _():
                    pltpu.async_copy(
                        tab_hbm.at[idxb.at[pl.ds((cB + 1) * CHUNK, CHUNK)]],
                        slabA, semA)

                @pl.when(cB + 1 >= SUPER)
                def _():
                    pltpu.make_async_copy(
                        idx_hbm.at[pl.ds(nsbase, STOK)], nidxb, nsemP).wait()
                    pltpu.make_async_copy(
                        q_hbm.at[pl.ds(nsbase, STOK)], nqb, nsemP).wait()
                    pltpu.async_copy(
                        tab_hbm.at[nidxb.at[pl.ds(0, CHUNK)]], slabA, semA)

                pltpu.make_async_copy(
                    tab_hbm.at[idxb.at[pl.ds(cB * CHUNK, CHUNK)]],
                    slabB, semB).wait()
                compute_chunk(cB, slabB, qb)
                return 0

            lax.fori_loop(0, SUPER // 2, pair_body, 0)
            pltpu.sync_copy(cxt_big, out_hbm.at[pl.ds(sbase, STOK)])

        base0 = wid * tpw
        pltpu.sync_copy(idx_hbm.at[pl.ds(base0, STOK)], idx0)
        pltpu.sync_copy(q_hbm.at[pl.ds(base0, STOK)], q0)
        pltpu.async_copy(tab_hbm.at[idx0.at[pl.ds(0, CHUNK)]], slabA, semA)

        def spair_body(sp, _):
            run_super(2 * sp, idx0, q0, idx1, q1, semP1)
            run_super(2 * sp + 1, idx1, q1, idx0, q0, semP0)
            return 0

        lax.fori_loop(0, n_super // 2, spair_body, 0)
        pltpu.make_async_copy(
            tab_hbm.at[idx0.at[pl.ds(0, CHUNK)]], slabA, semA).wait()

    return body(table, batch_idx, q)


def _loss_ce(x, cxt, labels, W1a, W1b, b1, W2, b2, blk=2048):
    T, D = x.shape
    V = W2.shape[1]
    nb = T // blk
    labels3 = labels.reshape(nb, 1, blk)

    def body(x_ref, c_ref, w1a_ref, w1b_ref, b1_ref, w2_ref, b2_ref,
             lab_ref, o_ref):
        i = pl.program_id(0)
        h = jnp.dot(x_ref[...], w1a_ref[...], preferred_element_type=F32)
        h += jnp.dot(c_ref[...], w1b_ref[...], preferred_element_type=F32)
        h = jnp.maximum(h + b1_ref[...], 0.0)
        logits = jnp.dot(h, w2_ref[...], preferred_element_type=F32) + b2_ref[...]
        m = jnp.max(logits, axis=-1, keepdims=True)
        lse = jnp.log(jnp.sum(jnp.exp(logits - m), axis=-1)) + m[:, 0]
        lab = lab_ref[0, 0, :]
        onehot = lax.broadcasted_iota(I32, (blk, V), 1) == lab[:, None]
        picked = jnp.sum(jnp.where(onehot, logits, 0.0), axis=-1)
        part = jnp.sum(lse - picked)

        @pl.when(i == 0)
        def _():
            o_ref[0, 0] = 0.0

        o_ref[0, 0] += part

    out = pl.pallas_call(
        body,
        grid=(nb,),
        in_specs=[
            pl.BlockSpec((blk, D), lambda i: (i, 0)),
            pl.BlockSpec((blk, LAT), lambda i: (i, 0)),
            pl.BlockSpec(W1a.shape, lambda i: (0, 0)),
            pl.BlockSpec(W1b.shape, lambda i: (0, 0)),
            pl.BlockSpec((1, W1a.shape[1]), lambda i: (0, 0)),
            pl.BlockSpec(W2.shape, lambda i: (0, 0)),
            pl.BlockSpec((1, V), lambda i: (0, 0)),
            pl.BlockSpec((1, 1, blk), lambda i: (i, 0, 0)),
        ],
        out_specs=pl.BlockSpec(memory_space=pltpu.SMEM),
        out_shape=jax.ShapeDtypeStruct((1, 1), F32),
    )(x, cxt, W1a, W1b, b1.reshape(1, -1), W2, b2.reshape(1, V), labels3)
    return out[0, 0]


def _loss_bce(x, cxt, labels, W1a, W1b, b1, W2, b2, blk=2048):
    T, D = x.shape
    nb = T // blk
    labels3 = labels.reshape(nb, 1, blk)

    def body(x_ref, c_ref, w1a_ref, w1b_ref, b1_ref, w2_ref, b2_ref,
             lab_ref, o_ref):
        i = pl.program_id(0)
        h = jnp.dot(x_ref[...], w1a_ref[...], preferred_element_type=F32)
        h += jnp.dot(c_ref[...], w1b_ref[...], preferred_element_type=F32)
        h = jnp.maximum(h + b1_ref[...], 0.0)
        s = jnp.dot(h, w2_ref[...], preferred_element_type=F32)[:, 0] + b2_ref[0, 0]
        y = lab_ref[0, 0, :].astype(F32)
        part = jnp.sum(
            jnp.maximum(s, 0.0) - s * y + jnp.log(1.0 + jnp.exp(-jnp.abs(s)))
        )

        @pl.when(i == 0)
        def _():
            o_ref[0, 0] = 0.0

        o_ref[0, 0] += part

    out = pl.pallas_call(
        body,
        grid=(nb,),
        in_specs=[
            pl.BlockSpec((blk, D), lambda i: (i, 0)),
            pl.BlockSpec((blk, LAT), lambda i: (i, 0)),
            pl.BlockSpec(W1a.shape, lambda i: (0, 0)),
            pl.BlockSpec(W1b.shape, lambda i: (0, 0)),
            pl.BlockSpec((1, W1a.shape[1]), lambda i: (0, 0)),
            pl.BlockSpec(W2.shape, lambda i: (0, 0)),
            pl.BlockSpec((1, 1), lambda i: (0, 0)),
            pl.BlockSpec((1, 1, blk), lambda i: (i, 0, 0)),
        ],
        out_specs=pl.BlockSpec(memory_space=pltpu.SMEM),
        out_shape=jax.ShapeDtypeStruct((1, 1), F32),
    )(x, cxt, W1a, W1b, b1.reshape(1, -1), W2, b2.reshape(1, 1), labels3)
    return out[0, 0]


def kernel(src_graph_vecs, topo_vecs, atom_vecs, bond_vecs,
           A_topo_W, A_topo_b, A_atom_W, A_atom_b, A_bond_W, A_bond_b,
           topo_W1, topo_b1, topo_W2, topo_b2,
           atom_W1, atom_b1, atom_W2, atom_b2,
           bond_W1, bond_b1, bond_W2, bond_b2,
           batch_idx_topo, batch_idx_atom, batch_idx_bond,
           topo_labels, atom_labels, bond_labels):
    B = src_graph_vecs.shape[0]
    table = src_graph_vecs.reshape(B, ROW)

    q_topo = _qproj(topo_vecs, A_topo_W, A_topo_b)
    q_atom = _qproj(atom_vecs, A_atom_W, A_atom_b)
    q_bond = _qproj(bond_vecs, A_bond_W, A_bond_b)

    c_topo = _attention_sc(table, batch_idx_topo, q_topo)
    c_atom = _attention_sc(table, batch_idx_atom, q_atom)
    c_bond = _attention_sc(table, batch_idx_bond, q_bond)

    Dt = topo_vecs.shape[1]
    Da = atom_vecs.shape[1]
    Db = bond_vecs.shape[1]
    lt = _loss_bce(topo_vecs, c_topo, topo_labels,
                   topo_W1[:Dt], topo_W1[Dt:], topo_b1, topo_W2, topo_b2)
    la = _loss_ce(atom_vecs, c_atom, atom_labels,
                  atom_W1[:Da], atom_W1[Da:], atom_b1, atom_W2, atom_b2)
    lb = _loss_ce(bond_vecs, c_bond, bond_labels,
                  bond_W1[:Db], bond_W1[Db:], bond_b1, bond_W2, bond_b2)

    return (lt + la + lb) / B


# speedup vs baseline: 1.4354x; 1.0051x over previous
import functools

import jax
import jax.numpy as jnp
from jax import lax
from jax.experimental import pallas as pl
from jax.experimental.pallas import tpu as pltpu
from jax.experimental.pallas import tpu_sc as plsc

F32 = jnp.float32
I32 = jnp.int32

NC = 2
NS = 16
NW = NC * NS
LANES = 16
CHUNK = 16
L_NODES = 32
LAT = 64
ROW = L_NODES * LAT


def _qproj(x, W, b, blk=4096):
    T, D = x.shape
    O = W.shape[1]

    def body(x_ref, w_ref, b_ref, o_ref):
        o_ref[...] = (
            jnp.dot(x_ref[...], w_ref[...], preferred_element_type=F32)
            + b_ref[...]
        )

    return pl.pallas_call(
        body,
        grid=(T // blk,),
        in_specs=[
            pl.BlockSpec((blk, D), lambda i: (i, 0)),
            pl.BlockSpec((D, O), lambda i: (0, 0)),
            pl.BlockSpec((1, O), lambda i: (0, 0)),
        ],
        out_specs=pl.BlockSpec((blk, O), lambda i: (i, 0)),
        out_shape=jax.ShapeDtypeStruct((T, O), F32),
    )(x, W, b.reshape(1, O))


SUPER = 8
STOK = SUPER * CHUNK


def _attention_sc(table, batch_idx, q):
    T = batch_idx.shape[0]
    tpw = T // NW
    n_super = tpw // STOK

    mesh = plsc.VectorSubcoreMesh(
        core_axis_name="c", subcore_axis_name="s",
        num_cores=NC, num_subcores=NS)

    @functools.partial(
        pl.kernel,
        out_type=jax.ShapeDtypeStruct((T, LAT), F32),
        mesh=mesh,
        compiler_params=pltpu.CompilerParams(needs_layout_passes=False),
        scratch_types=[
            pltpu.VMEM((STOK,), I32),
            pltpu.VMEM((STOK,), I32),
            pltpu.VMEM((STOK, LAT), F32),
            pltpu.VMEM((STOK, LAT), F32),
            pltpu.VMEM((STOK, LAT), F32),
            pltpu.VMEM((CHUNK, ROW), F32),
            pltpu.VMEM((CHUNK, ROW), F32),
            pltpu.SemaphoreType.DMA,
            pltpu.SemaphoreType.DMA,
            pltpu.SemaphoreType.DMA,
            pltpu.SemaphoreType.DMA,
        ],
    )
    def body(tab_hbm, idx_hbm, q_hbm, out_hbm, idx0, idx1, q0, q1, cxt_big,
             slabA, slabB, semA, semB, semP0, semP1):
        wid = lax.axis_index("s") * NC + lax.axis_index("c")
        lane = lax.iota(I32, LANES)

        def compute_chunk(c, slab, q_big):
            def one_token(t):
                row = c * CHUNK + t
                qr = [q_big[row, pl.ds(dc * 16, 16)] for dc in range(4)]

                sv = [jnp.zeros((LANES,), F32), jnp.zeros((LANES,), F32)]
                for h in range(2):
                    for j in range(16):
                        l = h * 16 + j
                        acc = slab[t, pl.ds(l * LAT, 16)] * qr[0]
                        for dc in range(1, 4):
                            acc += slab[t, pl.ds(l * LAT + dc * 16, 16)] * qr[dc]
                        s = jnp.sum(acc)
                        sv[h] = jnp.where(lane == j, s, sv[h])

                m = jnp.full((LANES,), jnp.max(jnp.maximum(sv[0], sv[1])), F32)
                e = [jnp.exp(sv[0] - m), jnp.exp(sv[1] - m)]
                inv = 1.0 / jnp.full((LANES,), jnp.sum(e[0] + e[1]), F32)

                w = [jnp.zeros((LANES,), F32) for _ in range(4)]
                for h in range(2):
                    for j in range(16):
                        l = h * 16 + j
                        pj = e[h][j]
                        for dc in range(4):
                            w[dc] += pj * slab[t, pl.ds(l * LAT + dc * 16, 16)]
                for dc in range(4):
                    cxt_big[row, pl.ds(dc * 16, 16)] = w[dc] * inv

            def tok_pair(p, _):
                one_token(2 * p)
                one_token(2 * p + 1)
                return 0

            lax.fori_loop(0, CHUNK // 2, tok_pair, 0)

        def run_super(s, idxb, qb, nidxb, nqb, nsemP):
            sbase = wid * tpw + s * STOK
            ns = s + 1
            nsbase = wid * tpw + jnp.where(ns < n_super, ns, 0) * STOK
            pltpu.async_copy(idx_hbm.at[pl.ds(nsbase, STOK)], nidxb, nsemP)
            pltpu.async_copy(q_hbm.at[pl.ds(nsbase, STOK)], nqb, nsemP)

            def pair_body(p, _):
                cA = 2 * p
                cB = 2 * p + 1
                pltpu.async_copy(
                    tab_hbm.at[idxb.at[pl.ds(cB * CHUNK, CHUNK)]],
                    slabB, semB)
                pltpu.make_async_copy(
                    tab_hbm.at[idxb.at[pl.ds(cA * CHUNK, CHUNK)]],
                    slabA, semA).wait()
                compute_chunk(cA, slabA, qb)

                @pl.when(cB + 1 < SUPER)
                def _():
                    pltpu.async_copy(
                        tab_hbm.at[idxb.at[pl.ds((cB + 1) * CHUNK, CHUNK)]],
                        slabA, semA)

                @pl.when(cB + 1 >= SUPER)
                def _():
                    pltpu.make_async_copy(
                        idx_hbm.at[pl.ds(nsbase, STOK)], nidxb, nsemP).wait()
                    pltpu.make_async_copy(
                        q_hbm.at[pl.ds(nsbase, STOK)], nqb, nsemP).wait()
                    pltpu.async_copy(
                        tab_hbm.at[nidxb.at[pl.ds(0, CHUNK)]], slabA, semA)

                pltpu.make_async_copy(
                    tab_hbm.at[idxb.at[pl.ds(cB * CHUNK, CHUNK)]],
                    slabB, semB).wait()
                compute_chunk(cB, slabB, qb)
                return 0

            lax.fori_loop(0, SUPER // 2, pair_body, 0)
            pltpu.sync_copy(cxt_big, out_hbm.at[pl.ds(sbase, STOK)])

        base0 = wid * tpw
        pltpu.sync_copy(idx_hbm.at[pl.ds(base0, STOK)], idx0)
        pltpu.sync_copy(q_hbm.at[pl.ds(base0, STOK)], q0)
        pltpu.async_copy(tab_hbm.at[idx0.at[pl.ds(0, CHUNK)]], slabA, semA)

        def spair_body(sp, _):
            run_super(2 * sp, idx0, q0, idx1, q1, semP1)
            run_super(2 * sp + 1, idx1, q1, idx0, q0, semP0)
            return 0

        lax.fori_loop(0, n_super // 2, spair_body, 0)
        pltpu.make_async_copy(
            tab_hbm.at[idx0.at[pl.ds(0, CHUNK)]], slabA, semA).wait()

    return body(table, batch_idx, q)


def _loss_ce(x, cxt, labels, W1a, W1b, b1, W2, b2, blk=4096):
    T, D = x.shape
    V = W2.shape[1]
    nb = T // blk
    labels3 = labels.reshape(nb, 1, blk)

    def body(x_ref, c_ref, w1a_ref, w1b_ref, b1_ref, w2_ref, b2_ref,
             lab_ref, o_ref):
        i = pl.program_id(0)
        h = jnp.dot(x_ref[...], w1a_ref[...], preferred_element_type=F32)
        h += jnp.dot(c_ref[...], w1b_ref[...], preferred_element_type=F32)
        h = jnp.maximum(h + b1_ref[...], 0.0)
        logits = jnp.dot(h, w2_ref[...], preferred_element_type=F32) + b2_ref[...]
        m = jnp.max(logits, axis=-1, keepdims=True)
        lse = jnp.log(jnp.sum(jnp.exp(logits - m), axis=-1)) + m[:, 0]
        lab = lab_ref[0, 0, :]
        onehot = lax.broadcasted_iota(I32, (blk, V), 1) == lab[:, None]
        picked = jnp.sum(jnp.where(onehot, logits, 0.0), axis=-1)
        part = jnp.sum(lse - picked)

        @pl.when(i == 0)
        def _():
            o_ref[0, 0] = 0.0

        o_ref[0, 0] += part

    out = pl.pallas_call(
        body,
        grid=(nb,),
        in_specs=[
            pl.BlockSpec((blk, D), lambda i: (i, 0)),
            pl.BlockSpec((blk, LAT), lambda i: (i, 0)),
            pl.BlockSpec(W1a.shape, lambda i: (0, 0)),
            pl.BlockSpec(W1b.shape, lambda i: (0, 0)),
            pl.BlockSpec((1, W1a.shape[1]), lambda i: (0, 0)),
            pl.BlockSpec(W2.shape, lambda i: (0, 0)),
            pl.BlockSpec((1, V), lambda i: (0, 0)),
            pl.BlockSpec((1, 1, blk), lambda i: (i, 0, 0)),
        ],
        out_specs=pl.BlockSpec(memory_space=pltpu.SMEM),
        out_shape=jax.ShapeDtypeStruct((1, 1), F32),
    )(x, cxt, W1a, W1b, b1.reshape(1, -1), W2, b2.reshape(1, V), labels3)
    return out[0, 0]


def _loss_bce(x, cxt, labels, W1a, W1b, b1, W2, b2, blk=4096):
    T, D = x.shape
    nb = T // blk
    labels3 = labels.reshape(nb, 1, blk)

    def body(x_ref, c_ref, w1a_ref, w1b_ref, b1_ref, w2_ref, b2_ref,
             lab_ref, o_ref):
        i = pl.program_id(0)
        h = jnp.dot(x_ref[...], w1a_ref[...], preferred_element_type=F32)
        h += jnp.dot(c_ref[...], w1b_ref[...], preferred_element_type=F32)
        h = jnp.maximum(h + b1_ref[...], 0.0)
        s = jnp.dot(h, w2_ref[...], preferred_element_type=F32)[:, 0] + b2_ref[0, 0]
        y = lab_ref[0, 0, :].astype(F32)
        part = jnp.sum(
            jnp.maximum(s, 0.0) - s * y + jnp.log(1.0 + jnp.exp(-jnp.abs(s)))
        )

        @pl.when(i == 0)
        def _():
            o_ref[0, 0] = 0.0

        o_ref[0, 0] += part

    out = pl.pallas_call(
        body,
        grid=(nb,),
        in_specs=[
            pl.BlockSpec((blk, D), lambda i: (i, 0)),
            pl.BlockSpec((blk, LAT), lambda i: (i, 0)),
            pl.BlockSpec(W1a.shape, lambda i: (0, 0)),
            pl.BlockSpec(W1b.shape, lambda i: (0, 0)),
            pl.BlockSpec((1, W1a.shape[1]), lambda i: (0, 0)),
            pl.BlockSpec(W2.shape, lambda i: (0, 0)),
            pl.BlockSpec((1, 1), lambda i: (0, 0)),
            pl.BlockSpec((1, 1, blk), lambda i: (i, 0, 0)),
        ],
        out_specs=pl.BlockSpec(memory_space=pltpu.SMEM),
        out_shape=jax.ShapeDtypeStruct((1, 1), F32),
    )(x, cxt, W1a, W1b, b1.reshape(1, -1), W2, b2.reshape(1, 1), labels3)
    return out[0, 0]


def kernel(src_graph_vecs, topo_vecs, atom_vecs, bond_vecs,
           A_topo_W, A_topo_b, A_atom_W, A_atom_b, A_bond_W, A_bond_b,
           topo_W1, topo_b1, topo_W2, topo_b2,
           atom_W1, atom_b1, atom_W2, atom_b2,
           bond_W1, bond_b1, bond_W2, bond_b2,
           batch_idx_topo, batch_idx_atom, batch_idx_bond,
           topo_labels, atom_labels, bond_labels):
    B = src_graph_vecs.shape[0]
    table = src_graph_vecs.reshape(B, ROW)

    q_topo = _qproj(topo_vecs, A_topo_W, A_topo_b)
    q_atom = _qproj(atom_vecs, A_atom_W, A_atom_b)
    q_bond = _qproj(bond_vecs, A_bond_W, A_bond_b)

    c_topo = _attention_sc(table, batch_idx_topo, q_topo)
    c_atom = _attention_sc(table, batch_idx_atom, q_atom)
    c_bond = _attention_sc(table, batch_idx_bond, q_bond)

    Dt = topo_vecs.shape[1]
    Da = atom_vecs.shape[1]
    Db = bond_vecs.shape[1]
    lt = _loss_bce(topo_vecs, c_topo, topo_labels,
                   topo_W1[:Dt], topo_W1[Dt:], topo_b1, topo_W2, topo_b2)
    la = _loss_ce(atom_vecs, c_atom, atom_labels,
                  atom_W1[:Da], atom_W1[Da:], atom_b1, atom_W2, atom_b2)
    lb = _loss_ce(bond_vecs, c_bond, bond_labels,
                  bond_W1[:Db], bond_W1[Db:], bond_b1, bond_W2, bond_b2)

    return (lt + la + lb) / B
